# Initial kernel scaffold; baseline (speedup 1.0000x reference)
import jax
import jax.numpy as jnp
from jax.experimental import pallas as pl


def kernel(x, boxes, box_labels, memory, W_ff, b_ff, W_mp, b_mp, Wq, bq, Wk, bk, Wv, bv, Wo, bo, W_fuse, b_fuse, W_head, b_head):
    raise NotImplementedError("write your pallas kernel here")


# speedup vs baseline: 3.1414x; 3.1414x over previous
import functools

import jax
import jax.numpy as jnp
import numpy as np
from jax import lax
from jax.experimental import pallas as pl
from jax.experimental.pallas import tpu as pltpu
from jax.experimental.pallas import tpu_sc as plsc

D = 2048
H = 16
DK = D // H
M = 2048
K = 32
B = 64
NR = 196
NC = 100
NB = 5000
THR = 0.5

_INV_SQRT_DK = 1.0 / np.sqrt(DK)
_INT_MIN = np.int32(-2147483648)


def _mean_body(x_ref, o_ref):
    acc = None
    for t in range(25):
        lo = 8 * t
        n = 8 if t < 24 else 4
        s = x_ref[:, lo, :]
        for j in range(1, n):
            s = s + x_ref[:, lo + j, :]
        acc = s if acc is None else acc + s
    o_ref[...] = acc / np.float32(NR)


def _mean(x):
    bt = 8
    return pl.pallas_call(
        _mean_body,
        grid=(B // bt,),
        in_specs=[pl.BlockSpec((bt, NR, D), lambda i: (i, 0, 0))],
        out_specs=pl.BlockSpec((bt, D), lambda i: (i, 0)),
        out_shape=jax.ShapeDtypeStruct((B, D), jnp.float32),
    )(x)


def _xq_body(xm_ref, wff_ref, bff_ref, wq_ref, bq_ref, xf_ref, q_ref):
    xf = jnp.dot(xm_ref[...], wff_ref[...], preferred_element_type=jnp.float32)
    xf = xf + bff_ref[...]
    xf_ref[...] = xf
    q = jnp.dot(xf, wq_ref[...], preferred_element_type=jnp.float32)
    q_ref[...] = q + bq_ref[...]


def _xq(xm, w_ff, b_ff, wq, bq):
    return pl.pallas_call(
        _xq_body,
        out_shape=(
            jax.ShapeDtypeStruct((B, D), jnp.float32),
            jax.ShapeDtypeStruct((B, D), jnp.float32),
        ),
    )(xm, w_ff, b_ff, wq, bq)


def _memproj_body(m_ref, w_ref, b_ref, o_ref):
    o_ref[...] = (
        jnp.dot(m_ref[...], w_ref[...], preferred_element_type=jnp.float32)
        + b_ref[...]
    )


def _memproj(memory, w_mp, b_mp):
    mt = 256
    return pl.pallas_call(
        _memproj_body,
        grid=(M // mt,),
        in_specs=[
            pl.BlockSpec((mt, D), lambda i: (i, 0)),
            pl.BlockSpec((D, D), lambda i: (0, 0)),
            pl.BlockSpec((1, D), lambda i: (0, 0)),
        ],
        out_specs=pl.BlockSpec((mt, D), lambda i: (i, 0)),
        out_shape=jax.ShapeDtypeStruct((M, D), jnp.float32),
    )(memory, w_mp, b_mp)


def _kv_body(m_ref, wk_ref, bk_ref, wv_ref, bv_ref, k_ref, v_ref):
    m = m_ref[...]
    k_ref[...] = (
        jnp.dot(m, wk_ref[...], preferred_element_type=jnp.float32) + bk_ref[...]
    )
    v_ref[...] = (
        jnp.dot(m, wv_ref[...], preferred_element_type=jnp.float32) + bv_ref[...]
    )


def _kv(mem, wk, bk, wv, bv):
    mt = 256
    wspec = pl.BlockSpec((D, D), lambda i: (0, 0))
    bspec = pl.BlockSpec((1, D), lambda i: (0, 0))
    mspec = pl.BlockSpec((mt, D), lambda i: (i, 0))
    return pl.pallas_call(
        _kv_body,
        grid=(M // mt,),
        in_specs=[mspec, wspec, bspec, wspec, bspec],
        out_specs=(mspec, mspec),
        out_shape=(
            jax.ShapeDtypeStruct((M, D), jnp.float32),
            jax.ShapeDtypeStruct((M, D), jnp.float32),
        ),
    )(mem, wk, bk, wv, bv)


def _attn_body(q_ref, k_ref, v_ref, o_ref):
    q = q_ref[...]
    k = k_ref[...]
    s = lax.dot_general(
        q, k, (((1,), (1,)), ((), ())), preferred_element_type=jnp.float32
    ) * np.float32(_INV_SQRT_DK)

    ibits = lax.bitcast_convert_type(s, jnp.int32)
    key = jnp.where(ibits < 0, ibits ^ np.int32(0x7FFFFFFF), ibits)

    t = jnp.full((B, 1), _INT_MIN, jnp.int32)
    for bit in range(31, -1, -1):
        cand = t ^ np.int32(np.uint32(1 << bit))
        cnt = jnp.sum((key >= cand).astype(jnp.int32), axis=1, keepdims=True)
        t = jnp.where(cnt >= K, cand, t)

    gt = key > t
    eq = key == t
    n_gt = jnp.sum(gt.astype(jnp.int32), axis=1, keepdims=True)
    need = K - n_gt
    col = lax.broadcasted_iota(jnp.int32, (B, M), 1)
    idxv = jnp.where(eq, col, jnp.int32(M))
    u = jnp.zeros((B, 1), jnp.int32)
    for bit in range(11, -1, -1):
        cand = u | np.int32((1 << bit) - 1)
        cnt = jnp.sum((idxv <= cand).astype(jnp.int32), axis=1, keepdims=True)
        u = jnp.where(cnt >= need, u, u | np.int32(1 << bit))

    sel = gt | (eq & (col <= u))

    mx = jnp.max(s, axis=1, keepdims=True)
    w = jnp.where(sel, jnp.exp(s - mx), np.float32(0.0))
    p = w / jnp.sum(w, axis=1, keepdims=True)
    o_ref[...] = jnp.dot(p, v_ref[...], preferred_element_type=jnp.float32)


def _attn(q, k, v):
    return pl.pallas_call(
        _attn_body,
        grid=(H,),
        in_specs=[
            pl.BlockSpec((B, DK), lambda h: (0, h)),
            pl.BlockSpec((M, DK), lambda h: (0, h)),
            pl.BlockSpec((M, DK), lambda h: (0, h)),
        ],
        out_specs=pl.BlockSpec((B, DK), lambda h: (0, h)),
        out_shape=jax.ShapeDtypeStruct((B, D), jnp.float32),
    )(q, k, v)


def _resp_body(o_ref, wo_ref, bo_ref, r_ref):
    r_ref[...] = (
        jnp.dot(o_ref[...], wo_ref[...], preferred_element_type=jnp.float32)
        + bo_ref[...]
    )


def _resp(out, wo, bo):
    return pl.pallas_call(
        _resp_body,
        out_shape=jax.ShapeDtypeStruct((B, D), jnp.float32),
    )(out, wo, bo)


def _fuse_body(xf_ref, r_ref, wf_ref, bf_ref, z_ref):
    zc = jnp.concatenate([xf_ref[...], r_ref[...]], axis=1)
    z_ref[...] = (
        jnp.dot(zc, wf_ref[...], preferred_element_type=jnp.float32) + bf_ref[...]
    )


def _fuse(xf, resp, wf, bf):
    return pl.pallas_call(
        _fuse_body,
        out_shape=jax.ShapeDtypeStruct((B, D), jnp.float32),
    )(xf, resp, wf, bf)


def _head_body(z_ref, u_ref, wh_ref, bh_ref, logits_ref, probs_ref, sel_ref):
    g = np.float32(0.5) * z_ref[...] * u_ref[...]
    logits = (
        jnp.dot(g, wh_ref[...], preferred_element_type=jnp.float32) + bh_ref[...]
    )
    probs = jax.nn.sigmoid(logits)
    logits_ref[...] = logits
    probs_ref[...] = probs
    sel_ref[...] = (probs > np.float32(THR)).astype(jnp.int32)


def _head(z, u, wh, bh):
    return pl.pallas_call(
        _head_body,
        out_shape=(
            jax.ShapeDtypeStruct((B, NC), jnp.float32),
            jax.ShapeDtypeStruct((B, NC), jnp.float32),
            jax.ShapeDtypeStruct((B, NC), jnp.int32),
        ),
    )(z, u, wh, bh)


_SC_CORES = 2
_SC_SUBCORES = 16
_NBP = 5120
_PERW = _NBP // (_SC_CORES * _SC_SUBCORES)


def _box_gather(sel_flat, bcol, labels):
    mesh = plsc.VectorSubcoreMesh(
        core_axis_name="c",
        subcore_axis_name="s",
        num_cores=_SC_CORES,
        num_subcores=_SC_SUBCORES,
    )

    @functools.partial(
        pl.kernel,
        out_type=jax.ShapeDtypeStruct((_NBP,), jnp.int32),
        mesh=mesh,
        compiler_params=pltpu.CompilerParams(needs_layout_passes=False),
        scratch_types=[
            pltpu.VMEM((B * NC,), jnp.int32),
            pltpu.VMEM((_PERW,), jnp.int32),
            pltpu.VMEM((_PERW,), jnp.int32),
            pltpu.VMEM((_PERW,), jnp.int32),
        ],
    )
    def kern(sel_hbm, b_hbm, l_hbm, out_hbm, table_v, bv, lv, ov):
        wid = lax.axis_index("s") * _SC_CORES + lax.axis_index("c")
        base = wid * _PERW
        pltpu.sync_copy(sel_hbm, table_v)
        pltpu.sync_copy(b_hbm.at[pl.ds(base, _PERW)], bv)
        pltpu.sync_copy(l_hbm.at[pl.ds(base, _PERW)], lv)
        for j in range(_PERW // 16):
            bb = bv[pl.ds(j * 16, 16)]
            ll = lv[pl.ds(j * 16, 16)]
            idx = bb * np.int32(NC) + ll
            ov[pl.ds(j * 16, 16)] = plsc.load_gather(table_v, [idx])
        pltpu.sync_copy(ov, out_hbm.at[pl.ds(base, _PERW)])

    return kern(sel_flat, bcol, labels)


def kernel(x, boxes, box_labels, memory, W_ff, b_ff, W_mp, b_mp, Wq, bq,
           Wk, bk, Wv, bv, Wo, bo, W_fuse, b_fuse, W_head, b_head):
    b_ff2 = b_ff.reshape(1, D)
    bq2 = bq.reshape(1, D)
    b_mp2 = b_mp.reshape(1, D)
    bk2 = bk.reshape(1, D)
    bv2 = bv.reshape(1, D)
    bo2 = bo.reshape(1, D)
    bf2 = b_fuse.reshape(1, D)
    bh2 = b_head.reshape(1, NC)

    xm = _mean(x)
    xf, q = _xq(xm, W_ff, b_ff2, Wq, bq2)
    mem = _memproj(memory, W_mp, b_mp2)
    kk, vv = _kv(mem, Wk, bk2, Wv, bv2)
    out = _attn(q, kk, vv)
    resp = _resp(out, Wo, bo2)
    z = _fuse(xf, resp, W_fuse, bf2)
    u = lax.erfc(-z * np.float32(np.sqrt(0.5)))
    logits, probs, sel = _head(z, u, W_head, bh2)

    sel_flat = sel.reshape(B * NC)
    bcol = jnp.pad(boxes[:, 0].astype(jnp.int32), (0, _NBP - NB))
    lab = jnp.pad(box_labels.astype(jnp.int32), (0, _NBP - NB))
    bm = _box_gather(sel_flat, bcol, lab)[:NB].astype(bool)
    return (logits, probs, bm)


# speedup vs baseline: 3.2041x; 1.0200x over previous
import functools

import jax
import jax.numpy as jnp
import numpy as np
from jax import lax
from jax.experimental import pallas as pl
from jax.experimental.pallas import tpu as pltpu
from jax.experimental.pallas import tpu_sc as plsc

D = 2048
H = 16
DK = D // H
M = 2048
K = 32
B = 64
NR = 196
NC = 100
NB = 5000
THR = 0.5

_INV_SQRT_DK = 1.0 / np.sqrt(DK)
_INT_MIN = np.int32(-2147483648)


def _meanxq_body(x_ref, wff_ref, bff_ref, wq_ref, bq_ref,
                 xf_ref, q_ref, acc_ref):
    i = pl.program_id(0)
    acc = None
    for t in range(25):
        lo = 8 * t
        n = 8 if t < 24 else 4
        s = x_ref[:, lo, :]
        for j in range(1, n):
            s = s + x_ref[:, lo + j, :]
        acc = s if acc is None else acc + s
    acc_ref[pl.ds(i * 8, 8), :] = acc / np.float32(NR)

    @pl.when(i == B // 8 - 1)
    def _tail():
        xf = jnp.dot(acc_ref[...], wff_ref[...],
                     preferred_element_type=jnp.float32)
        xf = xf + bff_ref[...]
        xf_ref[...] = xf
        q = jnp.dot(xf, wq_ref[...], preferred_element_type=jnp.float32)
        q_ref[...] = q + bq_ref[...]


def _meanxq(x, w_ff, b_ff, wq, bq):
    bt = 8
    wspec = pl.BlockSpec((D, D), lambda i: (0, 0))
    bspec = pl.BlockSpec((1, D), lambda i: (0, 0))
    return pl.pallas_call(
        _meanxq_body,
        grid=(B // bt,),
        in_specs=[
            pl.BlockSpec((bt, NR, D), lambda i: (i, 0, 0)),
            wspec, bspec, wspec, bspec,
        ],
        out_specs=(
            pl.BlockSpec((B, D), lambda i: (0, 0)),
            pl.BlockSpec((B, D), lambda i: (0, 0)),
        ),
        out_shape=(
            jax.ShapeDtypeStruct((B, D), jnp.float32),
            jax.ShapeDtypeStruct((B, D), jnp.float32),
        ),
        scratch_shapes=[pltpu.VMEM((B, D), jnp.float32)],
    )(x, w_ff, b_ff, wq, bq)


def _mkv_body(m_ref, wp_ref, bp_ref, wk_ref, bk_ref, wv_ref, bv_ref,
              k_ref, v_ref):
    mem = (
        jnp.dot(m_ref[...], wp_ref[...], preferred_element_type=jnp.float32)
        + bp_ref[...]
    )
    k_ref[...] = (
        jnp.dot(mem, wk_ref[...], preferred_element_type=jnp.float32) + bk_ref[...]
    )
    v_ref[...] = (
        jnp.dot(mem, wv_ref[...], preferred_element_type=jnp.float32) + bv_ref[...]
    )


def _mkv(memory, w_mp, b_mp, wk, bk, wv, bv):
    mt = 128
    wspec = pl.BlockSpec((D, D), lambda i: (0, 0))
    bspec = pl.BlockSpec((1, D), lambda i: (0, 0))
    mspec = pl.BlockSpec((mt, D), lambda i: (i, 0))
    return pl.pallas_call(
        _mkv_body,
        grid=(M // mt,),
        in_specs=[mspec, wspec, bspec, wspec, bspec, wspec, bspec],
        out_specs=(mspec, mspec),
        out_shape=(
            jax.ShapeDtypeStruct((M, D), jnp.float32),
            jax.ShapeDtypeStruct((M, D), jnp.float32),
        ),
    )(memory, w_mp, b_mp, wk, bk, wv, bv)


def _attn_body(q_ref, k_ref, v_ref, o_ref):
    q = q_ref[...]
    k = k_ref[...]
    s = lax.dot_general(
        q, k, (((1,), (1,)), ((), ())), preferred_element_type=jnp.float32
    ) * np.float32(_INV_SQRT_DK)

    ibits = lax.bitcast_convert_type(s, jnp.int32)
    key = jnp.where(ibits < 0, ibits ^ np.int32(0x7FFFFFFF), ibits)

    t = jnp.full((B, 1), _INT_MIN, jnp.int32)
    for bit in range(31, -1, -1):
        cand = t ^ np.int32(np.uint32(1 << bit))
        cnt = jnp.sum((key >= cand).astype(jnp.int32), axis=1, keepdims=True)
        t = jnp.where(cnt >= K, cand, t)

    gt = key > t
    eq = key == t
    n_gt = jnp.sum(gt.astype(jnp.int32), axis=1, keepdims=True)
    need = K - n_gt
    col = lax.broadcasted_iota(jnp.int32, (B, M), 1)
    idxv = jnp.where(eq, col, jnp.int32(M))
    u = jnp.zeros((B, 1), jnp.int32)
    for bit in range(11, -1, -1):
        cand = u | np.int32((1 << bit) - 1)
        cnt = jnp.sum((idxv <= cand).astype(jnp.int32), axis=1, keepdims=True)
        u = jnp.where(cnt >= need, u, u | np.int32(1 << bit))

    sel = gt | (eq & (col <= u))

    mx = jnp.max(s, axis=1, keepdims=True)
    w = jnp.where(sel, jnp.exp(s - mx), np.float32(0.0))
    p = w / jnp.sum(w, axis=1, keepdims=True)
    o_ref[...] = jnp.dot(p, v_ref[...], preferred_element_type=jnp.float32)


def _attn(q, k, v):
    return pl.pallas_call(
        _attn_body,
        grid=(H,),
        in_specs=[
            pl.BlockSpec((B, DK), lambda h: (0, h)),
            pl.BlockSpec((M, DK), lambda h: (0, h)),
            pl.BlockSpec((M, DK), lambda h: (0, h)),
        ],
        out_specs=pl.BlockSpec((B, DK), lambda h: (0, h)),
        out_shape=jax.ShapeDtypeStruct((B, D), jnp.float32),
    )(q, k, v)


def _respfuse_body(o_ref, wo_ref, bo_ref, xf_ref, wf_ref, bf_ref, z_ref):
    r = (
        jnp.dot(o_ref[...], wo_ref[...], preferred_element_type=jnp.float32)
        + bo_ref[...]
    )
    zc = jnp.concatenate([xf_ref[...], r], axis=1)
    z_ref[...] = (
        jnp.dot(zc, wf_ref[...], preferred_element_type=jnp.float32) + bf_ref[...]
    )


def _respfuse(out, wo, bo, xf, wf, bf):
    return pl.pallas_call(
        _respfuse_body,
        out_shape=jax.ShapeDtypeStruct((B, D), jnp.float32),
    )(out, wo, bo, xf, wf, bf)


def _head_body(z_ref, u_ref, wh_ref, bh_ref, logits_ref, probs_ref, sel_ref):
    g = np.float32(0.5) * z_ref[...] * u_ref[...]
    logits = (
        jnp.dot(g, wh_ref[...], preferred_element_type=jnp.float32) + bh_ref[...]
    )
    probs = jax.nn.sigmoid(logits)
    logits_ref[...] = logits
    probs_ref[...] = probs
    sel_ref[...] = (probs > np.float32(THR)).astype(jnp.int32)


def _head(z, u, wh, bh):
    return pl.pallas_call(
        _head_body,
        out_shape=(
            jax.ShapeDtypeStruct((B, NC), jnp.float32),
            jax.ShapeDtypeStruct((B, NC), jnp.float32),
            jax.ShapeDtypeStruct((B, NC), jnp.int32),
        ),
    )(z, u, wh, bh)


_SC_CORES = 2
_SC_SUBCORES = 16
_NBP = 5120
_PERW = _NBP // (_SC_CORES * _SC_SUBCORES)


def _box_gather(sel_flat, bcol, labels):
    mesh = plsc.VectorSubcoreMesh(
        core_axis_name="c",
        subcore_axis_name="s",
        num_cores=_SC_CORES,
        num_subcores=_SC_SUBCORES,
    )

    @functools.partial(
        pl.kernel,
        out_type=jax.ShapeDtypeStruct((_NBP,), jnp.int32),
        mesh=mesh,
        compiler_params=pltpu.CompilerParams(needs_layout_passes=False),
        scratch_types=[
            pltpu.VMEM((B * NC,), jnp.int32),
            pltpu.VMEM((_PERW,), jnp.int32),
            pltpu.VMEM((_PERW,), jnp.int32),
            pltpu.VMEM((_PERW,), jnp.int32),
        ],
    )
    def kern(sel_hbm, b_hbm, l_hbm, out_hbm, table_v, bv, lv, ov):
        wid = lax.axis_index("s") * _SC_CORES + lax.axis_index("c")
        base = wid * _PERW
        pltpu.sync_copy(sel_hbm, table_v)
        pltpu.sync_copy(b_hbm.at[pl.ds(base, _PERW)], bv)
        pltpu.sync_copy(l_hbm.at[pl.ds(base, _PERW)], lv)
        for j in range(_PERW // 16):
            bb = bv[pl.ds(j * 16, 16)]
            ll = lv[pl.ds(j * 16, 16)]
            idx = bb * np.int32(NC) + ll
            ov[pl.ds(j * 16, 16)] = plsc.load_gather(table_v, [idx])
        pltpu.sync_copy(ov, out_hbm.at[pl.ds(base, _PERW)])

    return kern(sel_flat, bcol, labels)


def kernel(x, boxes, box_labels, memory, W_ff, b_ff, W_mp, b_mp, Wq, bq,
           Wk, bk, Wv, bv, Wo, bo, W_fuse, b_fuse, W_head, b_head):
    b_ff2 = b_ff.reshape(1, D)
    bq2 = bq.reshape(1, D)
    b_mp2 = b_mp.reshape(1, D)
    bk2 = bk.reshape(1, D)
    bv2 = bv.reshape(1, D)
    bo2 = bo.reshape(1, D)
    bf2 = b_fuse.reshape(1, D)
    bh2 = b_head.reshape(1, NC)

    xf, q = _meanxq(x, W_ff, b_ff2, Wq, bq2)
    kk, vv = _mkv(memory, W_mp, b_mp2, Wk, bk2, Wv, bv2)
    out = _attn(q, kk, vv)
    z = _respfuse(out, Wo, bo2, xf, W_fuse, bf2)
    u = lax.erfc(-z * np.float32(np.sqrt(0.5)))
    logits, probs, sel = _head(z, u, W_head, bh2)

    sel_flat = sel.reshape(B * NC)
    bcol = jnp.pad(boxes[:, 0].astype(jnp.int32), (0, _NBP - NB))
    lab = jnp.pad(box_labels.astype(jnp.int32), (0, _NBP - NB))
    bm = _box_gather(sel_flat, bcol, lab)[:NB].astype(bool)
    return (logits, probs, bm)
